# Initial kernel scaffold; baseline (speedup 1.0000x reference)
import jax
import jax.numpy as jnp
from jax.experimental import pallas as pl


def kernel(x, neigh, W1, b1, W2, b2):
    raise NotImplementedError("write your pallas kernel here")


# speedup vs baseline: 1.5173x; 1.5173x over previous
import functools

import jax
import jax.numpy as jnp
from jax import lax
from jax.experimental import pallas as pl
from jax.experimental.pallas import tpu as pltpu
from jax.experimental.pallas import tpu_sc as plsc

_NC = 2
_NS = 16
_NW = _NC * _NS
_NPAD = 10240


def _gather_sum(table, idx_chunks, bc, nchunk, deg):
    n_pad, f = table.shape
    ec = bc * deg
    bw = bc * nchunk
    mesh = plsc.VectorSubcoreMesh(core_axis_name="c", subcore_axis_name="s")

    @functools.partial(
        pl.kernel,
        out_type=jax.ShapeDtypeStruct((n_pad, f), jnp.float32),
        mesh=mesh,
        scratch_types=[
            pltpu.VMEM((nchunk, ec), jnp.int32),
            pltpu.VMEM((ec, f), jnp.float32),
            pltpu.VMEM((ec, f), jnp.float32),
            pltpu.VMEM((bc, f), jnp.float32),
            pltpu.VMEM((bc, f), jnp.float32),
            pltpu.SemaphoreType.DMA,
            pltpu.SemaphoreType.DMA,
            pltpu.SemaphoreType.DMA,
            pltpu.SemaphoreType.DMA,
        ],
    )
    def k(table_hbm, idx_hbm, out_hbm,
          idx_v, buf0, buf1, ob0, ob1, sg0, sg1, so0, so1):
        cid = lax.axis_index("c")
        sid = lax.axis_index("s")
        wid = sid * _NC + cid
        base = wid * bw
        pltpu.sync_copy(idx_hbm.at[wid], idx_v)
        bufs, obs = (buf0, buf1), (ob0, ob1)
        sgs, sos = (sg0, sg1), (so0, so1)

        pltpu.async_copy(table_hbm.at[idx_v.at[0]], buf0, sg0)

        def pair(p, carry):
            for q in range(2):
                c = p * 2 + q
                buf, ob, sg, so = bufs[q], obs[q], sgs[q], sos[q]

                @pl.when(c + 1 < nchunk)
                def _():
                    pltpu.async_copy(table_hbm.at[idx_v.at[c + 1]],
                                     bufs[1 - q], sgs[1 - q])

                pltpu.make_async_copy(table_hbm.at[idx_v.at[c]], buf,
                                      sg).wait()

                @pl.when(c >= 2)
                def _():
                    pltpu.make_async_copy(
                        ob, out_hbm.at[pl.ds(base, bc)], so).wait()

                def red(b, carry2):
                    e0 = b * deg
                    for g in range(f // 16):
                        sl = pl.ds(g * 16, 16)
                        acc = buf[e0, sl]
                        for j in range(1, deg):
                            acc = acc + buf[e0 + j, sl]
                        ob[b, sl] = acc
                    return carry2

                lax.fori_loop(0, bc, red, 0)
                pltpu.async_copy(ob, out_hbm.at[pl.ds(base + c * bc, bc)],
                                 so)
            return carry

        lax.fori_loop(0, nchunk // 2, pair, 0)
        pltpu.make_async_copy(ob0, out_hbm.at[pl.ds(base, bc)], so0).wait()
        pltpu.make_async_copy(ob1, out_hbm.at[pl.ds(base, bc)], so1).wait()

    return k(table, idx_chunks)


def _sage_linear(a, s, w_self, w_neigh, b, inv_deg):
    m, k = a.shape
    h = w_self.shape[0]
    bm = 512
    dn = (((1,), (1,)), ((), ()))

    def body(a_ref, s_ref, wa_ref, wn_ref, b_ref, o_ref):
        acc = lax.dot_general(a_ref[...], wa_ref[...], dn,
                              preferred_element_type=jnp.float32)
        acc += lax.dot_general(s_ref[...] * inv_deg, wn_ref[...], dn,
                               preferred_element_type=jnp.float32)
        o_ref[...] = jnp.maximum(acc + b_ref[...], 0.0)

    return pl.pallas_call(
        body,
        grid=(m // bm,),
        in_specs=[
            pl.BlockSpec((bm, k), lambda i: (i, 0)),
            pl.BlockSpec((bm, k), lambda i: (i, 0)),
            pl.BlockSpec((h, k), lambda i: (0, 0)),
            pl.BlockSpec((h, k), lambda i: (0, 0)),
            pl.BlockSpec((1, h), lambda i: (0, 0)),
        ],
        out_specs=pl.BlockSpec((bm, h), lambda i: (i, 0)),
        out_shape=jax.ShapeDtypeStruct((m, h), jnp.float32),
    )(a, s, w_self, w_neigh, b)


def kernel(x, neigh, W1, b1, W2, b2):
    n, d = x.shape
    deg = neigh.shape[1]
    h_dim = W1.shape[0]
    pad = _NPAD - n

    x_pad = jnp.pad(x, ((0, pad), (0, 0)))
    neigh_pad = jnp.pad(neigh, ((0, pad), (0, 0)))
    bc1, nch1 = 8, _NPAD // (_NW * 8)
    bc2, nch2 = 4, _NPAD // (_NW * 4)
    idx1 = neigh_pad.reshape(_NW, nch1, bc1 * deg)
    idx2 = neigh_pad.reshape(_NW, nch2, bc2 * deg)

    inv_deg = 1.0 / deg
    s1 = _gather_sum(x_pad, idx1, bc1, nch1, deg)
    h1 = _sage_linear(x_pad, s1, W1[:, :d], W1[:, d:], b1.reshape(1, -1),
                      inv_deg)
    s2 = _gather_sum(h1, idx2, bc2, nch2, deg)
    out = _sage_linear(h1, s2, W2[:, :h_dim], W2[:, h_dim:],
                       b2.reshape(1, -1), inv_deg)
    return out[:n]
